# baseline (speedup 1.0000x reference)
import functools

import jax
import jax.numpy as jnp
from jax import lax
from jax.experimental import pallas as pl
from jax.experimental.pallas import tpu as pltpu
from jax.experimental.pallas import tpu_sc as plsc

_LANES = 16
_CHUNK = 2048
_IDX_ROW = 128


@functools.lru_cache(maxsize=None)
def _build_sampler(N, H, W):
    info = plsc.get_sparse_core_info()
    nc, ns = info.num_cores, info.num_subcores
    nw = nc * ns
    assert N % (nw * _CHUNK) == 0
    npt = N // nw
    nchunk = npt // _CHUNK
    G = _CHUNK // _LANES
    R = (4 * _CHUNK) // _IDX_ROW
    C = _CHUNK
    fw = float(W - 1)
    fh = float(H - 1)

    mesh = plsc.VectorSubcoreMesh(core_axis_name="c", subcore_axis_name="s")

    @functools.partial(
        pl.kernel, mesh=mesh,
        out_type=jax.ShapeDtypeStruct((N,), jnp.float32),
        scratch_types=[
            pltpu.VMEM((C,), jnp.float32),
            pltpu.VMEM((C,), jnp.float32),
            pltpu.VMEM((C,), jnp.float32),
            pltpu.VMEM((C,), jnp.float32),
            pltpu.VMEM((4 * C,), jnp.int32),
            pltpu.VMEM((4 * C,), jnp.float32),
            pltpu.VMEM((C,), jnp.float32),
            pltpu.VMEM((4 * _LANES,), jnp.float32),
            pltpu.SemaphoreType.DMA,
        ],
    )
    def sampler(xs_hbm, ys_hbm, img_hbm, sc_hbm, out_hbm,
                xsv, ysv, wxv, wyv, idxv, valv, outv, scv, sem):
        wid = lax.axis_index("s") * nc + lax.axis_index("c")
        base = wid * npt

        pltpu.sync_copy(sc_hbm, scv)
        aminx = scv[pl.ds(0, _LANES)]
        aminy = scv[pl.ds(_LANES, _LANES)]
        invdx = scv[pl.ds(2 * _LANES, _LANES)]
        invdy = scv[pl.ds(3 * _LANES, _LANES)]

        def build(g, _):
            o = g * _LANES
            gx = xsv[pl.ds(o, _LANES)]
            gy = ysv[pl.ds(o, _LANES)]
            xn = (gx - aminx) * 2.0 * invdx - 1.0
            yn = (gy - aminy) * 2.0 * invdy - 1.0
            fx = (xn + 1.0) * 0.5 * fw
            fy = (yn + 1.0) * 0.5 * fh
            ix = jnp.minimum(jnp.maximum(fx.astype(jnp.int32), 0), W - 2)
            iy = jnp.minimum(jnp.maximum(fy.astype(jnp.int32), 0), H - 2)
            wxv[pl.ds(o, _LANES)] = fx - ix.astype(jnp.float32)
            wyv[pl.ds(o, _LANES)] = fy - iy.astype(jnp.float32)
            i00 = iy * W + ix
            idxv[pl.ds(o, _LANES)] = i00
            idxv[pl.ds(C + o, _LANES)] = i00 + 1
            idxv[pl.ds(2 * C + o, _LANES)] = i00 + W
            idxv[pl.ds(3 * C + o, _LANES)] = i00 + (W + 1)
            return _

        def fire(r, _):
            o = r * _IDX_ROW
            pltpu.async_copy(img_hbm.at[idxv.at[pl.ds(o, _IDX_ROW)]],
                             valv.at[pl.ds(o, _IDX_ROW)], sem)
            return _

        def combine(g, _):
            o = g * _LANES
            v00 = valv[pl.ds(o, _LANES)]
            v01 = valv[pl.ds(C + o, _LANES)]
            v10 = valv[pl.ds(2 * C + o, _LANES)]
            v11 = valv[pl.ds(3 * C + o, _LANES)]
            wx1 = wxv[pl.ds(o, _LANES)]
            wy1 = wyv[pl.ds(o, _LANES)]
            wx0 = 1.0 - wx1
            wy0 = 1.0 - wy1
            outv[pl.ds(o, _LANES)] = (v00 * wy0 * wx0 + v01 * wy0 * wx1
                                      + v10 * wy1 * wx0 + v11 * wy1 * wx1)
            return _

        def chunk(c, _):
            cb = pl.multiple_of(base + c * C, 8)
            pltpu.sync_copy(xs_hbm.at[pl.ds(cb, C)], xsv)
            pltpu.sync_copy(ys_hbm.at[pl.ds(cb, C)], ysv)
            lax.fori_loop(0, G, build, None)
            lax.fori_loop(0, R, fire, None)
            pltpu.make_async_copy(img_hbm.at[pl.ds(0, 4 * C)], valv, sem).wait()
            lax.fori_loop(0, G, combine, None)
            pltpu.sync_copy(outv, out_hbm.at[pl.ds(cb, C)])
            return _

        lax.fori_loop(0, nchunk, chunk, None)

    return sampler


def kernel(x, image, aabb_min, aabb_max):
    orig_shape = x.shape
    x_flat = x.reshape(-1, 2)
    N = x_flat.shape[0]
    H, W = image.shape[2], image.shape[3]
    inv = 1.0 / (aabb_max - aabb_min)
    ones = jnp.ones((_LANES,), jnp.float32)
    sc = jnp.concatenate([aabb_min[0] * ones, aabb_min[1] * ones,
                          inv[0] * ones, inv[1] * ones])
    xs = x_flat[:, 0]
    ys = x_flat[:, 1]
    out = _build_sampler(N, H, W)(xs, ys, image.reshape(H * W), sc)
    return out.reshape(orig_shape[:-1])


# speedup vs baseline: 1.3405x; 1.3405x over previous
import functools

import jax
import jax.numpy as jnp
from jax import lax
from jax.experimental import pallas as pl
from jax.experimental.pallas import tpu as pltpu
from jax.experimental.pallas import tpu_sc as plsc

_LANES = 16
_CHUNK = 2048
_IDX_ROW = 128


@functools.lru_cache(maxsize=None)
def _build_sampler(N, H, W):
    info = plsc.get_sparse_core_info()
    nc, ns = info.num_cores, info.num_subcores
    nw = nc * ns
    assert N % (nw * 2 * _CHUNK) == 0
    npt = N // nw
    nchunk = npt // _CHUNK
    G = _CHUNK // _LANES
    R = (4 * _CHUNK) // _IDX_ROW
    C = _CHUNK
    fw = float(W - 1)
    fh = float(H - 1)

    mesh = plsc.VectorSubcoreMesh(core_axis_name="c", subcore_axis_name="s")

    buf_types = [
        pltpu.VMEM((C,), jnp.float32),
        pltpu.VMEM((C,), jnp.float32),
        pltpu.VMEM((C,), jnp.float32),
        pltpu.VMEM((C,), jnp.float32),
        pltpu.VMEM((4 * C,), jnp.int32),
        pltpu.VMEM((4 * C,), jnp.float32),
        pltpu.VMEM((C,), jnp.float32),
        pltpu.SemaphoreType.DMA,
    ]

    @functools.partial(
        pl.kernel, mesh=mesh,
        out_type=jax.ShapeDtypeStruct((N,), jnp.float32),
        scratch_types=buf_types + buf_types
        + [pltpu.VMEM((4 * _LANES,), jnp.float32)],
    )
    def sampler(xs_hbm, ys_hbm, img_hbm, sc_hbm, out_hbm, *scratch):
        bufs = (scratch[0:8], scratch[8:16])
        scv = scratch[16]
        wid = lax.axis_index("s") * nc + lax.axis_index("c")
        base = wid * npt

        pltpu.sync_copy(sc_hbm, scv)
        aminx = scv[pl.ds(0, _LANES)]
        aminy = scv[pl.ds(_LANES, _LANES)]
        invdx = scv[pl.ds(2 * _LANES, _LANES)]
        invdy = scv[pl.ds(3 * _LANES, _LANES)]

        def load_build(cb, b):
            xsv, ysv, wxv, wyv, idxv, valv, outv, sem = bufs[b]
            pltpu.sync_copy(xs_hbm.at[pl.ds(cb, C)], xsv)
            pltpu.sync_copy(ys_hbm.at[pl.ds(cb, C)], ysv)

            @plsc.parallel_loop(0, G, unroll=4)
            def build(g):
                o = g * _LANES
                gx = xsv[pl.ds(o, _LANES)]
                gy = ysv[pl.ds(o, _LANES)]
                xn = (gx - aminx) * 2.0 * invdx - 1.0
                yn = (gy - aminy) * 2.0 * invdy - 1.0
                fx = (xn + 1.0) * 0.5 * fw
                fy = (yn + 1.0) * 0.5 * fh
                ix = jnp.minimum(jnp.maximum(fx.astype(jnp.int32), 0), W - 2)
                iy = jnp.minimum(jnp.maximum(fy.astype(jnp.int32), 0), H - 2)
                wxv[pl.ds(o, _LANES)] = fx - ix.astype(jnp.float32)
                wyv[pl.ds(o, _LANES)] = fy - iy.astype(jnp.float32)
                i00 = iy * W + ix
                idxv[pl.ds(o, _LANES)] = i00
                idxv[pl.ds(C + o, _LANES)] = i00 + 1
                idxv[pl.ds(2 * C + o, _LANES)] = i00 + W
                idxv[pl.ds(3 * C + o, _LANES)] = i00 + (W + 1)

            def fire(r, _):
                o = r * _IDX_ROW
                pltpu.async_copy(img_hbm.at[idxv.at[pl.ds(o, _IDX_ROW)]],
                                 valv.at[pl.ds(o, _IDX_ROW)], sem)
                return _

            lax.fori_loop(0, R, fire, None)

        def finish(cb, b):
            xsv, ysv, wxv, wyv, idxv, valv, outv, sem = bufs[b]
            pltpu.make_async_copy(img_hbm.at[pl.ds(0, 4 * C)], valv, sem).wait()

            @plsc.parallel_loop(0, G, unroll=4)
            def combine(g):
                o = g * _LANES
                v00 = valv[pl.ds(o, _LANES)]
                v01 = valv[pl.ds(C + o, _LANES)]
                v10 = valv[pl.ds(2 * C + o, _LANES)]
                v11 = valv[pl.ds(3 * C + o, _LANES)]
                wx1 = wxv[pl.ds(o, _LANES)]
                wy1 = wyv[pl.ds(o, _LANES)]
                wx0 = 1.0 - wx1
                wy0 = 1.0 - wy1
                outv[pl.ds(o, _LANES)] = (v00 * wy0 * wx0 + v01 * wy0 * wx1
                                          + v10 * wy1 * wx0 + v11 * wy1 * wx1)

            pltpu.sync_copy(outv, out_hbm.at[pl.ds(cb, C)])

        load_build(pl.multiple_of(base, 8), 0)

        def pair(i, _):
            c0 = 2 * i
            cb0 = pl.multiple_of(base + c0 * C, 8)
            cb1 = pl.multiple_of(base + c0 * C + C, 8)
            cb2 = pl.multiple_of(base + c0 * C + 2 * C, 8)
            load_build(cb1, 1)
            finish(cb0, 0)

            @pl.when(c0 + 2 < nchunk)
            def _more():
                load_build(cb2, 0)

            finish(cb1, 1)
            return _

        lax.fori_loop(0, nchunk // 2, pair, None)

    return sampler


def kernel(x, image, aabb_min, aabb_max):
    orig_shape = x.shape
    x_flat = x.reshape(-1, 2)
    N = x_flat.shape[0]
    H, W = image.shape[2], image.shape[3]
    inv = 1.0 / (aabb_max - aabb_min)
    ones = jnp.ones((_LANES,), jnp.float32)
    sc = jnp.concatenate([aabb_min[0] * ones, aabb_min[1] * ones,
                          inv[0] * ones, inv[1] * ones])
    xs = x_flat[:, 0]
    ys = x_flat[:, 1]
    out = _build_sampler(N, H, W)(xs, ys, image.reshape(H * W), sc)
    return out.reshape(orig_shape[:-1])


# speedup vs baseline: 1.3511x; 1.0079x over previous
import functools

import jax
import jax.numpy as jnp
from jax import lax
from jax.experimental import pallas as pl
from jax.experimental.pallas import tpu as pltpu
from jax.experimental.pallas import tpu_sc as plsc

_LANES = 16
_CHUNK = 4096
_IDX_ROW = 128


@functools.lru_cache(maxsize=None)
def _build_sampler(N, H, W):
    info = plsc.get_sparse_core_info()
    nc, ns = info.num_cores, info.num_subcores
    nw = nc * ns
    assert N % (nw * 2 * _CHUNK) == 0
    npt = N // nw
    nchunk = npt // _CHUNK
    G = _CHUNK // _LANES
    R = (4 * _CHUNK) // _IDX_ROW
    C = _CHUNK
    fw = float(W - 1)
    fh = float(H - 1)

    mesh = plsc.VectorSubcoreMesh(core_axis_name="c", subcore_axis_name="s")

    buf_types = [
        pltpu.VMEM((C,), jnp.float32),
        pltpu.VMEM((C,), jnp.float32),
        pltpu.VMEM((C,), jnp.float32),
        pltpu.VMEM((C,), jnp.float32),
        pltpu.VMEM((4 * C,), jnp.int32),
        pltpu.VMEM((4 * C,), jnp.float32),
        pltpu.VMEM((C,), jnp.float32),
        pltpu.SemaphoreType.DMA,
        pltpu.SemaphoreType.DMA,
    ]

    @functools.partial(
        pl.kernel, mesh=mesh,
        out_type=jax.ShapeDtypeStruct((N,), jnp.float32),
        scratch_types=buf_types + buf_types
        + [pltpu.VMEM((4 * _LANES,), jnp.float32)],
    )
    def sampler(xs_hbm, ys_hbm, img_hbm, sc_hbm, out_hbm, *scratch):
        bufs = (scratch[0:9], scratch[9:18])
        scv = scratch[18]
        wid = lax.axis_index("s") * nc + lax.axis_index("c")
        base = wid * npt

        pltpu.sync_copy(sc_hbm, scv)
        aminx = scv[pl.ds(0, _LANES)]
        aminy = scv[pl.ds(_LANES, _LANES)]
        invdx = scv[pl.ds(2 * _LANES, _LANES)]
        invdy = scv[pl.ds(3 * _LANES, _LANES)]

        def cb_of(c):
            return pl.multiple_of(base + c * C, 8)

        def issue_coords(cb, b):
            xsv, ysv = bufs[b][0], bufs[b][1]
            semx = bufs[b][8]
            pltpu.async_copy(xs_hbm.at[pl.ds(cb, C)], xsv, semx)
            pltpu.async_copy(ys_hbm.at[pl.ds(cb, C)], ysv, semx)

        def build_fire(cb, b):
            xsv, ysv, wxv, wyv, idxv, valv, outv, sem, semx = bufs[b]
            pltpu.make_async_copy(xs_hbm.at[pl.ds(cb, C)], xsv, semx).wait()
            pltpu.make_async_copy(ys_hbm.at[pl.ds(cb, C)], ysv, semx).wait()

            @plsc.parallel_loop(0, G, unroll=4)
            def build(g):
                o = g * _LANES
                gx = xsv[pl.ds(o, _LANES)]
                gy = ysv[pl.ds(o, _LANES)]
                xn = (gx - aminx) * 2.0 * invdx - 1.0
                yn = (gy - aminy) * 2.0 * invdy - 1.0
                fx = (xn + 1.0) * 0.5 * fw
                fy = (yn + 1.0) * 0.5 * fh
                ix = jnp.minimum(jnp.maximum(fx.astype(jnp.int32), 0), W - 2)
                iy = jnp.minimum(jnp.maximum(fy.astype(jnp.int32), 0), H - 2)
                wxv[pl.ds(o, _LANES)] = fx - ix.astype(jnp.float32)
                wyv[pl.ds(o, _LANES)] = fy - iy.astype(jnp.float32)
                i00 = iy * W + ix
                idxv[pl.ds(o, _LANES)] = i00
                idxv[pl.ds(C + o, _LANES)] = i00 + 1
                idxv[pl.ds(2 * C + o, _LANES)] = i00 + W
                idxv[pl.ds(3 * C + o, _LANES)] = i00 + (W + 1)

            def fire(r, _):
                o = r * _IDX_ROW
                pltpu.async_copy(img_hbm.at[idxv.at[pl.ds(o, _IDX_ROW)]],
                                 valv.at[pl.ds(o, _IDX_ROW)], sem)
                return _

            lax.fori_loop(0, R, fire, None)

        def finish(cb, b):
            xsv, ysv, wxv, wyv, idxv, valv, outv, sem, semx = bufs[b]
            pltpu.make_async_copy(img_hbm.at[pl.ds(0, 4 * C)], valv, sem).wait()

            @plsc.parallel_loop(0, G, unroll=4)
            def combine(g):
                o = g * _LANES
                v00 = valv[pl.ds(o, _LANES)]
                v01 = valv[pl.ds(C + o, _LANES)]
                v10 = valv[pl.ds(2 * C + o, _LANES)]
                v11 = valv[pl.ds(3 * C + o, _LANES)]
                wx1 = wxv[pl.ds(o, _LANES)]
                wy1 = wyv[pl.ds(o, _LANES)]
                wx0 = 1.0 - wx1
                wy0 = 1.0 - wy1
                outv[pl.ds(o, _LANES)] = (v00 * wy0 * wx0 + v01 * wy0 * wx1
                                          + v10 * wy1 * wx0 + v11 * wy1 * wx1)

            pltpu.sync_copy(outv, out_hbm.at[pl.ds(cb, C)])

        issue_coords(cb_of(0), 0)
        issue_coords(cb_of(1), 1)
        build_fire(cb_of(0), 0)

        def pair(i, _):
            c0 = 2 * i
            build_fire(cb_of(c0 + 1), 1)

            @pl.when(c0 + 2 < nchunk)
            def _pf0():
                issue_coords(cb_of(c0 + 2), 0)

            finish(cb_of(c0), 0)

            @pl.when(c0 + 2 < nchunk)
            def _bf0():
                build_fire(cb_of(c0 + 2), 0)

            @pl.when(c0 + 3 < nchunk)
            def _pf1():
                issue_coords(cb_of(c0 + 3), 1)

            finish(cb_of(c0 + 1), 1)
            return _

        lax.fori_loop(0, nchunk // 2, pair, None)

    return sampler


def kernel(x, image, aabb_min, aabb_max):
    orig_shape = x.shape
    x_flat = x.reshape(-1, 2)
    N = x_flat.shape[0]
    H, W = image.shape[2], image.shape[3]
    inv = 1.0 / (aabb_max - aabb_min)
    ones = jnp.ones((_LANES,), jnp.float32)
    sc = jnp.concatenate([aabb_min[0] * ones, aabb_min[1] * ones,
                          inv[0] * ones, inv[1] * ones])
    xs = x_flat[:, 0]
    ys = x_flat[:, 1]
    out = _build_sampler(N, H, W)(xs, ys, image.reshape(H * W), sc)
    return out.reshape(orig_shape[:-1])
